# baseline (speedup 1.0000x reference)
import jax
import jax.numpy as jnp
from jax import lax
from jax.experimental import pallas as pl
from jax.experimental.pallas import tpu as pltpu
from jax.experimental.pallas import tpu_sc as plsc

B = 4096
L = 16
NUM_CORES = 2
NUM_SUBCORES = 16
BPW = B // (NUM_CORES * NUM_SUBCORES)
HALF = BPW // 2


def _gather_body(params_hbm, cs_hbm, out_hbm, table_v, idx_v, out_v, sem_t, sem_i, sem_o):
    wid = lax.axis_index("s") * NUM_CORES + lax.axis_index("c")
    base = wid * BPW
    ct = pltpu.async_copy(params_hbm, table_v.at[pl.ds(0, 10)], sem_t)
    ci = pltpu.async_copy(cs_hbm.at[pl.ds(base, BPW)], idx_v, sem_i)
    ct.wait()
    ci.wait()
    table = table_v[...]
    for j in range(HALF // L):
        idx = idx_v[pl.ds(j * L, L)]
        out_v[pl.ds(j * L, L)] = table.at[idx].get(mode="promise_in_bounds")
    c0 = pltpu.async_copy(
        out_v.at[pl.ds(0, HALF)], out_hbm.at[pl.ds(base, HALF)], sem_o
    )
    for j in range(HALF // L, BPW // L):
        idx = idx_v[pl.ds(j * L, L)]
        out_v[pl.ds(j * L, L)] = table.at[idx].get(mode="promise_in_bounds")
    c1 = pltpu.async_copy(
        out_v.at[pl.ds(HALF, HALF)], out_hbm.at[pl.ds(base + HALF, HALF)], sem_o
    )
    c0.wait()
    c1.wait()


@jax.jit
def _run(params, cs):
    mesh = plsc.VectorSubcoreMesh(
        core_axis_name="c", subcore_axis_name="s", num_cores=NUM_CORES
    )
    f = pl.kernel(
        _gather_body,
        mesh=mesh,
        out_type=jax.ShapeDtypeStruct((B,), jnp.float32),
        scratch_types=[
            pltpu.VMEM((L,), jnp.float32),
            pltpu.VMEM((BPW,), jnp.int32),
            pltpu.VMEM((BPW,), jnp.float32),
            pltpu.SemaphoreType.DMA,
            pltpu.SemaphoreType.DMA,
            pltpu.SemaphoreType.DMA,
        ],
    )
    return f(params, cs)


def kernel(cs, xs, params):
    return _run(params, cs.astype(jnp.int32))


# speedup vs baseline: 1.0915x; 1.0915x over previous
import jax
import jax.numpy as jnp
from jax import lax
from jax.experimental import pallas as pl
from jax.experimental.pallas import tpu as pltpu
from jax.experimental.pallas import tpu_sc as plsc

B = 4096
L = 16
NUM_CORES = 1
NUM_SUBCORES = 16
BPW = B // (NUM_CORES * NUM_SUBCORES)
HALF = BPW // 2


def _gather_body(params_hbm, cs_hbm, out_hbm, table_v, idx_v, out_v, sem_t, sem_i, sem_o):
    wid = lax.axis_index("s") * NUM_CORES + lax.axis_index("c")
    base = wid * BPW
    ct = pltpu.async_copy(params_hbm, table_v.at[pl.ds(0, 10)], sem_t)
    ci = pltpu.async_copy(cs_hbm.at[pl.ds(base, BPW)], idx_v, sem_i)
    ct.wait()
    ci.wait()
    table = table_v[...]
    for j in range(HALF // L):
        idx = idx_v[pl.ds(j * L, L)]
        out_v[pl.ds(j * L, L)] = table.at[idx].get(mode="promise_in_bounds")
    c0 = pltpu.async_copy(
        out_v.at[pl.ds(0, HALF)], out_hbm.at[pl.ds(base, HALF)], sem_o
    )
    for j in range(HALF // L, BPW // L):
        idx = idx_v[pl.ds(j * L, L)]
        out_v[pl.ds(j * L, L)] = table.at[idx].get(mode="promise_in_bounds")
    c1 = pltpu.async_copy(
        out_v.at[pl.ds(HALF, HALF)], out_hbm.at[pl.ds(base + HALF, HALF)], sem_o
    )
    c0.wait()
    c1.wait()


@jax.jit
def _run(params, cs):
    mesh = plsc.VectorSubcoreMesh(
        core_axis_name="c", subcore_axis_name="s", num_cores=NUM_CORES
    )
    f = pl.kernel(
        _gather_body,
        mesh=mesh,
        out_type=jax.ShapeDtypeStruct((B,), jnp.float32),
        scratch_types=[
            pltpu.VMEM((L,), jnp.float32),
            pltpu.VMEM((BPW,), jnp.int32),
            pltpu.VMEM((BPW,), jnp.float32),
            pltpu.SemaphoreType.DMA,
            pltpu.SemaphoreType.DMA,
            pltpu.SemaphoreType.DMA,
        ],
    )
    return f(params, cs)


def kernel(cs, xs, params):
    return _run(params, cs.astype(jnp.int32))
